# baseline (speedup 1.0000x reference)
import functools

import jax
import jax.numpy as jnp
from jax import lax
from jax.experimental import pallas as pl
from jax.experimental.pallas import tpu as pltpu
from jax.experimental.pallas import tpu_sc as plsc

VOCAB = 100000
B = 4096
L = 200
DIM = 64
NROWS = B * L
IG = 128
NIG = NROWS // IG
NC, NS = 2, 16
NW = NC * NS
G = NIG // NW
CHUNK = 64
SLOTS = NROWS // CHUNK // NW
NBUF = 4
HALF = NBUF // 2


def _emb_body(x2, w3, out, idx_v, bufs, cbufs, gsem, osem):
    wid = lax.axis_index("s") * NC + lax.axis_index("c")
    gbase = wid * G
    rbase = wid * (SLOTS * CHUNK)

    pltpu.sync_copy(x2.at[pl.ds(gbase, G), :], idx_v)

    def g_start(b, gi):
        g = gi // 2
        off = (gi % 2) * CHUNK
        for k in range(CHUNK // 16):
            v = idx_v[g, pl.ds(off + k * 16, 16)]
            idx_v[g, pl.ds(off + k * 16, 16)] = jnp.where(
                v == 0, jnp.int32(VOCAB - 1), v - 1)
        pltpu.async_copy(
            w3.at[idx_v.at[g, pl.ds(off, CHUNK)]], bufs.at[b], gsem.at[b])

    def g_wait(b, gi):
        g = gi // 2
        off = (gi % 2) * CHUNK
        pltpu.make_async_copy(
            w3.at[idx_v.at[g, pl.ds(off, CHUNK)]], bufs.at[b],
            gsem.at[b]).wait()

    def compact(b):
        def crow(p, carry):
            for h in range(2):
                for c in range(DIM // 16):
                    cbufs[b, p, pl.ds(h * DIM + c * 16, 16)] = (
                        bufs[b, 2 * p + h, pl.ds(c * 16, 16)])
            return carry

        lax.fori_loop(0, CHUNK // 2, crow, 0)

    rbase2 = wid * (SLOTS * CHUNK // 2)

    def o_start(b, gi):
        off2 = pl.multiple_of(rbase2 + gi * (CHUNK // 2), CHUNK // 2)
        pltpu.async_copy(
            cbufs.at[b], out.at[pl.ds(off2, CHUNK // 2), :], osem.at[b])

    def o_wait(b, gi):
        off2 = pl.multiple_of(rbase2 + gi * (CHUNK // 2), CHUNK // 2)
        pltpu.make_async_copy(
            cbufs.at[b], out.at[pl.ds(off2, CHUNK // 2), :],
            osem.at[b]).wait()

    for b in range(HALF):
        g_start(b, b)
    for b in range(HALF):
        g_wait(b, b)
        compact(b)
        o_start(b, b)
        g_start(b + HALF, b + HALF)

    def steady(o, carry):
        g0 = HALF + o * NBUF
        for k in range(NBUF):
            gi = g0 + k
            b = (HALF + k) % NBUF
            bn = k % NBUF
            g_wait(b, gi)
            compact(b)
            o_start(b, gi)
            o_wait(bn, gi - HALF)
            g_start(bn, gi + HALF)
        return carry

    lax.fori_loop(0, (SLOTS - NBUF) // NBUF, steady, 0)

    for i in range(HALF):
        gi = SLOTS - HALF + i
        g_wait(HALF + i, gi)
        compact(HALF + i)
        o_start(HALF + i, gi)
    for b in range(NBUF):
        o_wait(b, SLOTS - NBUF + b)


_emb = functools.partial(
    pl.kernel,
    out_type=jax.ShapeDtypeStruct((NROWS // 2, 2 * DIM), jnp.float32),
    mesh=plsc.VectorSubcoreMesh(core_axis_name="c", subcore_axis_name="s"),
    compiler_params=pltpu.CompilerParams(use_tc_tiling_on_sc=True),
    scratch_types=[
        pltpu.VMEM((G, IG), jnp.int32),
        pltpu.VMEM((NBUF, CHUNK, 2 * DIM), jnp.float32),
        pltpu.VMEM((NBUF, CHUNK // 2, 2 * DIM), jnp.float32),
        pltpu.SemaphoreType.DMA((NBUF,)),
        pltpu.SemaphoreType.DMA((NBUF,)),
    ],
)(_emb_body)


def kernel(x, W):
    w2 = jnp.concatenate([W, jnp.zeros((1, DIM), jnp.float32)], axis=0)
    w3 = jnp.pad(w2, ((0, 0), (0, 2 * DIM - DIM)))
    out = _emb(x.reshape(NIG, IG), w3)
    return out.reshape(B, L, DIM)


# speedup vs baseline: 1.7455x; 1.7455x over previous
import functools

import jax
import jax.numpy as jnp
from jax import lax
from jax.experimental import pallas as pl
from jax.experimental.pallas import tpu as pltpu
from jax.experimental.pallas import tpu_sc as plsc

VOCAB = 100000
B = 4096
L = 200
DIM = 64
NROWS = B * L
IG = 128
NIG = NROWS // IG
NC, NS = 2, 16
NW = NC * NS
G = NIG // NW
CHUNK = 32
SLOTS = NROWS // CHUNK // NW
NBUF = 8
HALF = NBUF // 2


def _emb_body(x2, w3, out, idx_v, bufs, cbufs, gsem, osem):
    wid = lax.axis_index("s") * NC + lax.axis_index("c")
    gbase = wid * G
    rbase = wid * (SLOTS * CHUNK)

    pltpu.sync_copy(x2.at[pl.ds(gbase, G), :], idx_v)

    def g_start(b, gi):
        g = gi // (IG // CHUNK)
        off = (gi % (IG // CHUNK)) * CHUNK
        for k in range(CHUNK // 16):
            v = idx_v[g, pl.ds(off + k * 16, 16)]
            idx_v[g, pl.ds(off + k * 16, 16)] = jnp.where(
                v == 0, jnp.int32(VOCAB - 1), v - 1)
        pltpu.async_copy(
            w3.at[idx_v.at[g, pl.ds(off, CHUNK)]], bufs.at[b], gsem.at[b])

    def g_wait(b, gi):
        g = gi // (IG // CHUNK)
        off = (gi % (IG // CHUNK)) * CHUNK
        pltpu.make_async_copy(
            w3.at[idx_v.at[g, pl.ds(off, CHUNK)]], bufs.at[b],
            gsem.at[b]).wait()

    def compact(b):
        def crow(r, carry):
            for c in range(DIM // 16):
                cbufs[b, r, pl.ds(c * 16, 16)] = bufs[b, r, pl.ds(c * 16, 16)]
            return carry

        lax.fori_loop(0, CHUNK, crow, 0)

    def o_start(b, gi):
        pltpu.async_copy(
            cbufs.at[b], out.at[pl.ds(rbase + gi * CHUNK, CHUNK), :],
            osem.at[b])

    def o_wait(b, gi):
        pltpu.make_async_copy(
            cbufs.at[b], out.at[pl.ds(rbase + gi * CHUNK, CHUNK), :],
            osem.at[b]).wait()

    for b in range(HALF):
        g_start(b, b)
    for b in range(HALF):
        g_wait(b, b)
        compact(b)
        o_start(b, b)
        g_start(b + HALF, b + HALF)

    def steady(o, carry):
        g0 = HALF + o * NBUF
        for k in range(NBUF):
            gi = g0 + k
            b = (HALF + k) % NBUF
            bn = k % NBUF
            g_wait(b, gi)
            compact(b)
            o_start(b, gi)
            o_wait(bn, gi - HALF)
            g_start(bn, gi + HALF)
        return carry

    lax.fori_loop(0, (SLOTS - NBUF) // NBUF, steady, 0)

    for i in range(HALF):
        gi = SLOTS - HALF + i
        g_wait(HALF + i, gi)
        compact(HALF + i)
        o_start(HALF + i, gi)
    for b in range(NBUF):
        o_wait(b, SLOTS - NBUF + b)


_emb = functools.partial(
    pl.kernel,
    out_type=jax.ShapeDtypeStruct((NROWS, DIM), jnp.float32),
    mesh=plsc.VectorSubcoreMesh(core_axis_name="c", subcore_axis_name="s"),
    compiler_params=pltpu.CompilerParams(use_tc_tiling_on_sc=True),
    scratch_types=[
        pltpu.VMEM((G, IG), jnp.int32),
        pltpu.VMEM((NBUF, CHUNK, 2 * DIM), jnp.float32),
        pltpu.VMEM((NBUF, CHUNK, DIM), jnp.float32),
        pltpu.SemaphoreType.DMA((NBUF,)),
        pltpu.SemaphoreType.DMA((NBUF,)),
    ],
)(_emb_body)


def kernel(x, W):
    w2 = jnp.concatenate([W, jnp.zeros((1, DIM), jnp.float32)], axis=0)
    w3 = jnp.pad(w2, ((0, 0), (0, 2 * DIM - DIM)))
    out = _emb(x.reshape(NIG, IG), w3)
    return out.reshape(B, L, DIM)
